# baseline (speedup 1.0000x reference)
import math

import jax
import jax.numpy as jnp
from jax.experimental import pallas as pl

_TH2 = (0.75 * math.pi) ** 2


def _body(x_ref, o_ref):
    x = x_ref[...]
    s = x[:, 0] * x[:, 0] + x[:, 1] * x[:, 1] + x[:, 2] * x[:, 2]
    o_ref[...] = s > _TH2


def kernel(t, xi):
    B, D = xi.shape
    BLK = 8192
    return pl.pallas_call(
        _body,
        grid=(B // BLK,),
        in_specs=[pl.BlockSpec((BLK, D), lambda i: (i, 0))],
        out_specs=pl.BlockSpec((BLK,), lambda i: (i,)),
        out_shape=jax.ShapeDtypeStruct((B,), jnp.bool_),
    )(xi)


# speedup vs baseline: 2.0189x; 2.0189x over previous
import functools
import math

import jax
import jax.numpy as jnp
from jax import lax
from jax.experimental import pallas as pl
from jax.experimental.pallas import tpu as pltpu
from jax.experimental.pallas import tpu_sc as plsc

_TH2 = (0.75 * math.pi) ** 2


def _make_sc_kernel(B, D):
    info = plsc.get_sparse_core_info()
    NC, NS, L = info.num_cores, info.num_subcores, info.num_lanes
    NW = NC * NS
    rows_per_w = B // NW
    SLAB = 2048
    nslab = rows_per_w // SLAB
    mesh = plsc.VectorSubcoreMesh(core_axis_name="c", subcore_axis_name="s")

    @functools.partial(
        pl.kernel,
        out_type=jax.ShapeDtypeStruct((B,), jnp.int32),
        mesh=mesh,
        scratch_types=[
            pltpu.VMEM((SLAB * D,), jnp.float32),
            pltpu.VMEM((SLAB,), jnp.int32),
        ],
        compiler_params=pltpu.CompilerParams(needs_layout_passes=False),
    )
    def body(xi_hbm, out_hbm, buf, obuf):
        wid = lax.axis_index("s") * NC + lax.axis_index("c")
        base = wid * rows_per_w
        iota16 = lax.iota(jnp.int32, L) * D

        for s in range(nslab):
            r0 = base + s * SLAB
            pltpu.sync_copy(xi_hbm.at[pl.ds(r0 * D, SLAB * D)], buf)

            def inner(k, carry):
                row = k * L
                idx0 = row * D + iota16
                g0 = plsc.load_gather(buf, [idx0])
                g1 = plsc.load_gather(buf, [idx0 + 1])
                g2 = plsc.load_gather(buf, [idx0 + 2])
                v = g0 * g0 + g1 * g1 + g2 * g2
                obuf[pl.ds(row, L)] = (v > _TH2).astype(jnp.int32)
                return carry

            lax.fori_loop(0, SLAB // L, inner, 0)
            pltpu.sync_copy(obuf, out_hbm.at[pl.ds(r0, SLAB)])

    return body


def kernel(t, xi):
    B, D = xi.shape
    out_i32 = _make_sc_kernel(B, D)(jnp.reshape(xi, (-1,)))
    return out_i32.astype(jnp.bool_)


# speedup vs baseline: 24.1485x; 11.9613x over previous
import functools
import math

import jax
import jax.numpy as jnp
from jax import lax
from jax.experimental import pallas as pl
from jax.experimental.pallas import tpu as pltpu
from jax.experimental.pallas import tpu_sc as plsc

_TH2 = (0.75 * math.pi) ** 2


def _make_sc_kernel(B):
    info = plsc.get_sparse_core_info()
    NC, NS, L = info.num_cores, info.num_subcores, info.num_lanes
    NW = NC * NS
    NT = B // 128
    tiles_per_w = NT // NW
    CH = 128
    nchunk = tiles_per_w // CH
    mesh = plsc.VectorSubcoreMesh(core_axis_name="c", subcore_axis_name="s")

    @functools.partial(
        pl.kernel,
        out_type=jax.ShapeDtypeStruct((B,), jnp.int32),
        mesh=mesh,
        scratch_types=[
            pltpu.VMEM((CH, 128), jnp.float32),
            pltpu.VMEM((CH, 128), jnp.float32),
            pltpu.VMEM((CH, 128), jnp.float32),
            pltpu.VMEM((CH * 128,), jnp.int32),
        ],
        compiler_params=pltpu.CompilerParams(needs_layout_passes=False),
    )
    def body(v_hbm, out_hbm, b0, b1, b2, obuf):
        wid = lax.axis_index("s") * NC + lax.axis_index("c")
        tc_base = wid * tiles_per_w

        for ch in range(nchunk):
            tc0 = tc_base + ch * CH
            pltpu.sync_copy(v_hbm.at[pl.ds(tc0, CH), 0, :], b0)
            pltpu.sync_copy(v_hbm.at[pl.ds(tc0, CH), 1, :], b1)
            pltpu.sync_copy(v_hbm.at[pl.ds(tc0, CH), 2, :], b2)

            def inner(i, carry):
                j = i >> 3
                l0 = (i & 7) * L
                v0 = b0[j, pl.ds(l0, L)]
                v1 = b1[j, pl.ds(l0, L)]
                v2 = b2[j, pl.ds(l0, L)]
                s = v0 * v0 + v1 * v1 + v2 * v2
                obuf[pl.ds(i * L, L)] = (s > _TH2).astype(jnp.int32)
                return carry

            lax.fori_loop(0, CH * 8, inner, 0)
            pltpu.sync_copy(obuf, out_hbm.at[pl.ds(tc0 * 128, CH * 128)])

    return body


def kernel(t, xi):
    B, D = xi.shape
    v = jnp.reshape(jnp.transpose(xi), (2, 8, B // 128, 128))
    v = jnp.transpose(v, (0, 2, 1, 3))
    v = jnp.reshape(v, (B // 64, 8, 128))
    out_i32 = _make_sc_kernel(B)(v)
    return out_i32.astype(jnp.bool_)


# speedup vs baseline: 26.4915x; 1.0970x over previous
import functools
import math

import jax
import jax.numpy as jnp
from jax import lax
from jax.experimental import pallas as pl
from jax.experimental.pallas import tpu as pltpu
from jax.experimental.pallas import tpu_sc as plsc

_TH2 = (0.75 * math.pi) ** 2


def _make_sc_kernel(B):
    info = plsc.get_sparse_core_info()
    NC, NS, L = info.num_cores, info.num_subcores, info.num_lanes
    NW = NC * NS
    NT = B // 128
    tiles_per_w = NT // NW
    CH = 128
    nchunk = tiles_per_w // CH
    mesh = plsc.VectorSubcoreMesh(core_axis_name="c", subcore_axis_name="s")

    @functools.partial(
        pl.kernel,
        out_type=jax.ShapeDtypeStruct((B,), jnp.int32),
        mesh=mesh,
        scratch_types=[
            pltpu.VMEM((CH, 128), jnp.float32),
            pltpu.VMEM((CH, 128), jnp.float32),
            pltpu.VMEM((CH, 128), jnp.float32),
            pltpu.VMEM((CH * 128,), jnp.int32),
        ],
        compiler_params=pltpu.CompilerParams(needs_layout_passes=False),
    )
    def body(v_hbm, out_hbm, b0, b1, b2, obuf):
        wid = lax.axis_index("s") * NC + lax.axis_index("c")
        tc_base = wid * tiles_per_w

        for ch in range(nchunk):
            tc0 = tc_base + ch * CH
            pltpu.sync_copy(v_hbm.at[pl.ds(tc0, CH), 0, :], b0)
            pltpu.sync_copy(v_hbm.at[pl.ds(tc0, CH), 1, :], b1)
            pltpu.sync_copy(v_hbm.at[pl.ds(tc0, CH), 2, :], b2)

            @plsc.parallel_loop(0, CH * 8, unroll=8)
            def inner(i):
                j = i >> 3
                l0 = (i & 7) * L
                v0 = b0[j, pl.ds(l0, L)]
                v1 = b1[j, pl.ds(l0, L)]
                v2 = b2[j, pl.ds(l0, L)]
                s = v0 * v0 + v1 * v1 + v2 * v2
                obuf[pl.ds(i * L, L)] = (s > _TH2).astype(jnp.int32)

            pltpu.sync_copy(obuf, out_hbm.at[pl.ds(tc0 * 128, CH * 128)])

    return body


def kernel(t, xi):
    B, D = xi.shape
    v = jnp.reshape(jnp.transpose(xi), (2, 8, B // 128, 128))
    v = jnp.transpose(v, (0, 2, 1, 3))
    v = jnp.reshape(v, (B // 64, 8, 128))
    out_i32 = _make_sc_kernel(B)(v)
    return out_i32.astype(jnp.bool_)


# speedup vs baseline: 50.0073x; 1.8877x over previous
import math

import jax
import jax.numpy as jnp
from jax.experimental import pallas as pl
from jax.experimental.pallas import tpu as pltpu

_TH2 = (0.75 * math.pi) ** 2


def _tc_body(v_hbm, o_ref, b0, b1, b2, sem):
    i = pl.program_id(0)
    blk = o_ref.shape[0]
    r0 = i * blk
    c0 = pltpu.make_async_copy(v_hbm.at[pl.ds(r0, blk), 0, :], b0, sem)
    c1 = pltpu.make_async_copy(v_hbm.at[pl.ds(r0, blk), 1, :], b1, sem)
    c2 = pltpu.make_async_copy(v_hbm.at[pl.ds(r0, blk), 2, :], b2, sem)
    c0.start()
    c1.start()
    c2.start()
    c0.wait()
    c1.wait()
    c2.wait()
    v0 = b0[...]
    v1 = b1[...]
    v2 = b2[...]
    s = v0 * v0 + v1 * v1 + v2 * v2
    o_ref[...] = s > _TH2


def kernel(t, xi):
    B, D = xi.shape
    NT = B // 128
    v = jnp.reshape(jnp.transpose(xi), (2, 8, NT, 128))
    v = jnp.transpose(v, (0, 2, 1, 3))
    v = jnp.reshape(v, (2 * NT, 8, 128))
    BLK = 1024
    out2d = pl.pallas_call(
        _tc_body,
        grid=(NT // BLK,),
        in_specs=[pl.BlockSpec(memory_space=pl.ANY)],
        out_specs=pl.BlockSpec((BLK, 128), lambda i: (i, 0)),
        out_shape=jax.ShapeDtypeStruct((NT, 128), jnp.bool_),
        scratch_shapes=[
            pltpu.VMEM((BLK, 128), jnp.float32),
            pltpu.VMEM((BLK, 128), jnp.float32),
            pltpu.VMEM((BLK, 128), jnp.float32),
            pltpu.SemaphoreType.DMA,
        ],
    )(v)
    return jnp.reshape(out2d, (B,))
